# baseline (device time: 152223 ns/iter reference)
import jax
import jax.numpy as jnp
from jax import lax
from jax.experimental import pallas as pl
from jax.experimental.pallas import tpu as pltpu

T = 4096
D = 1024
CH = 512
N_MAX = T // CH
ROW = (8, 128)


UNROLL = 8


def _body(cnt_ref, order_ref, x_ref, out_ref, xs_ref, gat_sems, keep_sem,
          send_sems, recv_sems):
    my_x = lax.axis_index("x")
    my_y = lax.axis_index("y")
    my_z = lax.axis_index("z")
    peer = (my_x, my_y, 1 - my_z)
    cnt0 = cnt_ref[0]

    is0 = my_z == 0
    send_count = jnp.where(is0, T - cnt0, cnt0)
    keep_count = T - send_count
    n_send = (send_count + CH - 1) // CH
    n_keep = (keep_count + CH - 1) // CH
    dst_shift = jnp.where(is0, -cnt0, T - cnt0)

    def gather_rows(dst_ref, start, sem):
        def grp(g, _):
            base = start + g * UNROLL
            for u in range(UNROLL):
                j = base + u
                pltpu.make_async_copy(
                    x_ref.at[order_ref[j]], dst_ref.at[j], sem
                ).start()
            return 0

        lax.fori_loop(0, CH // UNROLL, grp, 0)

    def chunk_bytes_wait(sem, dst_ref):
        pltpu.make_async_copy(
            x_ref.at[pl.ds(0, CH)], dst_ref.at[pl.ds(0, CH)], sem
        ).wait()

    def send_chunk_start(i):
        return jnp.where(
            is0,
            jnp.maximum(T - (i + 1) * CH, cnt0),
            jnp.minimum(i * CH, cnt0 - CH),
        )

    for i in range(N_MAX):

        @pl.when(i < n_send)
        def _(i=i):
            gather_rows(xs_ref, send_chunk_start(i), gat_sems.at[i])

    for i in range(N_MAX):
        src_start = send_chunk_start(i)
        dst_start = src_start + dst_shift

        @pl.when(i < n_send)
        def _(i=i, src_start=src_start, dst_start=dst_start):
            chunk_bytes_wait(gat_sems.at[i], xs_ref)
            rdma = pltpu.make_async_remote_copy(
                src_ref=xs_ref.at[pl.ds(src_start, CH)],
                dst_ref=out_ref.at[pl.ds(dst_start, CH)],
                send_sem=send_sems.at[i],
                recv_sem=recv_sems.at[i],
                device_id=peer,
                device_id_type=pl.DeviceIdType.MESH,
            )
            rdma.start()

    for i in range(N_MAX):
        start = jnp.where(
            is0,
            jnp.minimum(i * CH, cnt0 - CH),
            jnp.maximum(T - (i + 1) * CH, cnt0),
        )

        @pl.when(i < n_keep)
        def _(start=start):
            gather_rows(out_ref, start, keep_sem)

    for i in range(N_MAX):

        @pl.when(i < n_keep)
        def _():
            chunk_bytes_wait(keep_sem, out_ref)

    for i in range(N_MAX):

        @pl.when(i < n_send)
        def _(i=i):
            rdma = pltpu.make_async_remote_copy(
                src_ref=xs_ref.at[pl.ds(0, CH)],
                dst_ref=out_ref.at[pl.ds(0, CH)],
                send_sem=send_sems.at[i],
                recv_sem=recv_sems.at[i],
                device_id=peer,
                device_id_type=pl.DeviceIdType.MESH,
            )
            rdma.wait_recv()

    for i in range(N_MAX):

        @pl.when(i < n_send)
        def _(i=i):
            rdma = pltpu.make_async_remote_copy(
                src_ref=xs_ref.at[pl.ds(0, CH)],
                dst_ref=out_ref.at[pl.ds(0, CH)],
                send_sem=send_sems.at[i],
                recv_sem=recv_sems.at[i],
                device_id=peer,
                device_id_type=pl.DeviceIdType.MESH,
            )
            rdma.wait_send()


def kernel(x, dest):
    order = jnp.argsort(dest, stable=True).astype(jnp.int32)
    cnt0 = jnp.sum(dest == 0).astype(jnp.int32).reshape((1,))

    out = pl.pallas_call(
        _body,
        out_shape=jax.ShapeDtypeStruct((T, *ROW), jnp.float32),
        in_specs=[
            pl.BlockSpec(memory_space=pltpu.SMEM),
            pl.BlockSpec(memory_space=pltpu.SMEM),
            pl.BlockSpec(memory_space=pltpu.VMEM),
        ],
        out_specs=pl.BlockSpec(memory_space=pltpu.VMEM),
        scratch_shapes=[
            pltpu.VMEM((T, *ROW), jnp.float32),
            pltpu.SemaphoreType.DMA((N_MAX,)),
            pltpu.SemaphoreType.DMA,
            pltpu.SemaphoreType.DMA((N_MAX,)),
            pltpu.SemaphoreType.DMA((N_MAX,)),
        ],
    )(cnt0, order, x.reshape(T, *ROW))
    return out.reshape(T, D)


# device time: 137310 ns/iter; 1.1086x vs baseline; 1.1086x over previous
import jax
import jax.numpy as jnp
from jax import lax
from jax.experimental import pallas as pl
from jax.experimental.pallas import tpu as pltpu

T = 4096
D = 1024
G = 8
N_GRP = T // G
ROW = (8, 128)


def _body(cnt_ref, order_ref, x_ref, out_ref, send_sem, recv_sem, keep_sem):
    my_x = lax.axis_index("x")
    my_y = lax.axis_index("y")
    my_z = lax.axis_index("z")
    peer = (my_x, my_y, 1 - my_z)
    cnt0 = cnt_ref[0]

    is0 = my_z == 0
    send_count = jnp.where(is0, T - cnt0, cnt0)
    keep_count = T - send_count
    n_sgrp = (send_count + G - 1) // G
    n_kgrp = (keep_count + G - 1) // G
    dst_shift = jnp.where(is0, -cnt0, T - cnt0)

    def send_grp_start(g):
        return jnp.where(
            is0,
            jnp.maximum(T - (g + 1) * G, cnt0),
            jnp.minimum(g * G, cnt0 - G),
        )

    def keep_grp_start(g):
        return jnp.where(
            is0,
            jnp.minimum(g * G, cnt0 - G),
            jnp.maximum(T - (g + 1) * G, cnt0),
        )

    def send_grp(g, _):
        j0 = send_grp_start(g)
        for u in range(G):
            j = j0 + u
            pltpu.make_async_remote_copy(
                src_ref=x_ref.at[order_ref[j]],
                dst_ref=out_ref.at[j + dst_shift],
                send_sem=send_sem,
                recv_sem=recv_sem,
                device_id=peer,
                device_id_type=pl.DeviceIdType.MESH,
            ).start()
        return 0

    lax.fori_loop(0, n_sgrp, send_grp, 0)

    def keep_grp(g, _):
        j0 = keep_grp_start(g)
        for u in range(G):
            j = j0 + u
            pltpu.make_async_copy(
                x_ref.at[order_ref[j]], out_ref.at[j], keep_sem
            ).start()
        return 0

    lax.fori_loop(0, n_kgrp, keep_grp, 0)

    def wait_n(sem, n):
        desc = pltpu.make_async_copy(
            x_ref.at[pl.ds(0, G)], out_ref.at[pl.ds(0, G)], sem
        )
        lax.fori_loop(0, n, lambda i, _: (desc.wait(), 0)[1], 0)

    wait_n(keep_sem, n_kgrp)
    wait_n(recv_sem, n_sgrp)
    wait_n(send_sem, n_sgrp)


def kernel(x, dest):
    order = jnp.argsort(dest, stable=True).astype(jnp.int32)
    cnt0 = jnp.sum(dest == 0).astype(jnp.int32).reshape((1,))

    out = pl.pallas_call(
        _body,
        out_shape=jax.ShapeDtypeStruct((T, *ROW), jnp.float32),
        in_specs=[
            pl.BlockSpec(memory_space=pltpu.SMEM),
            pl.BlockSpec(memory_space=pltpu.SMEM),
            pl.BlockSpec(memory_space=pltpu.VMEM),
        ],
        out_specs=pl.BlockSpec(memory_space=pltpu.VMEM),
        scratch_shapes=[
            pltpu.SemaphoreType.DMA,
            pltpu.SemaphoreType.DMA,
            pltpu.SemaphoreType.DMA,
        ],
    )(cnt0, order, x.reshape(T, *ROW))
    return out.reshape(T, D)
